# baseline (speedup 1.0000x reference)
import jax
import jax.numpy as jnp
from jax import lax
from jax.experimental import pallas as pl
from jax.experimental.pallas import tpu as pltpu

WORD = 256
EMB = 16
H = WORD * EMB
T = 256
NB = H // T
KS = 1
HK = H // KS


WPC = HK // EMB


def _lstm_body(*refs):
    idx_ref, table_ref = refs[0], refs[1]
    w_refs = refs[2:2 + 3 * KS]
    bi_ih, bg_ih, bo_ih, bi_hh, bg_hh, bo_hh = refs[2 + 3 * KS:8 + 3 * KS]
    h1_ref, h2_ref, c_ref, x_s = refs[8 + 3 * KS:]
    j = pl.program_id(0)

    @pl.when(j == 0)
    def _():
        v_iota = lax.broadcasted_iota(jnp.int32, (WORD, WORD), 0)
        onehot_t = (v_iota == idx_ref[...][None, :]).astype(jnp.float32)
        emb = lax.dot_general(
            onehot_t, table_ref[...], (((0,), (0,)), ((), ())),
            preferred_element_type=jnp.float32)
        sel = (lax.broadcasted_iota(jnp.int32, (EMB, HK), 0)
               == lax.broadcasted_iota(jnp.int32, (EMB, HK), 1) % EMB
               ).astype(jnp.float32)
        mask = (lax.broadcasted_iota(jnp.int32, (WPC, HK), 0)
                == lax.broadcasted_iota(jnp.int32, (WPC, HK), 1) // EMB
                ).astype(jnp.float32)
        ones = jnp.ones((1, WPC), jnp.float32)
        for ks in range(KS):
            g = lax.dot_general(
                emb[ks * WPC:(ks + 1) * WPC, :], sel,
                (((1,), (0,)), ((), ())),
                preferred_element_type=jnp.float32)
            xc = lax.dot_general(
                ones, g * mask, (((1,), (0,)), ((), ())),
                preferred_element_type=jnp.float32)
            x_s[0, ks * HK:(ks + 1) * HK] = xc[0, :]

    x = x_s[...]
    dn = (((1,), (1,)), ((), ()))

    def bias(ref):
        return ref[...].reshape(1, T)

    def mv(gate):
        acc = None
        for ks in range(KS):
            part = lax.dot_general(
                x[:, ks * HK:(ks + 1) * HK], w_refs[gate * KS + ks][...],
                dn, preferred_element_type=jnp.float32)
            acc = part if acc is None else acc + part
        return acc

    gi = mv(0) + bias(bi_ih) + bias(bi_hh)
    gg = mv(1) + bias(bg_ih) + bias(bg_hh)
    go = mv(2) + bias(bo_ih) + bias(bo_hh)
    i = jax.nn.sigmoid(gi)
    g = jnp.tanh(gg)
    o = jax.nn.sigmoid(go)
    c = i * g
    h = o * jnp.tanh(c)
    h1_ref[...] = h
    h2_ref[...] = h
    c_ref[...] = c


def _lstm_pallas(idx, table, W_ih, b_ih1, b_hh1):
    w_spec = lambda off, ks: pl.BlockSpec(
        (T, HK), lambda j, off=off, ks=ks: (j + off, ks))
    b_spec = lambda off: pl.BlockSpec((T,), lambda j, off=off: (j + off,))
    in_specs = [
        pl.BlockSpec((WORD,), lambda j: (0,)),
        pl.BlockSpec((WORD, EMB), lambda j: (0, 0)),
    ]
    in_specs += [w_spec(off, ks)
                 for off in (0, 2 * NB, 3 * NB) for ks in range(KS)]
    in_specs += [b_spec(0), b_spec(2 * NB), b_spec(3 * NB)] * 2
    out_specs = [pl.BlockSpec((1, T), lambda j: (0, j))] * 3
    out_shape = [jax.ShapeDtypeStruct((1, H), jnp.float32)] * 3
    return pl.pallas_call(
        _lstm_body,
        grid=(NB,),
        in_specs=in_specs,
        out_specs=out_specs,
        out_shape=out_shape,
        scratch_shapes=[pltpu.VMEM((1, H), jnp.float32)],
    )(idx, table, *([W_ih] * (3 * KS)),
      b_ih1, b_ih1, b_ih1, b_hh1, b_hh1, b_hh1)


def kernel(input, table, W_ih, W_hh, b_ih, b_hh):
    del W_hh
    h1, h2, c = _lstm_pallas(input.astype(jnp.int32), table, W_ih,
                             b_ih, b_hh)
    return (h1.reshape(1, 1, H), h2.reshape(1, 1, H), c.reshape(1, 1, H))


# speedup vs baseline: 1.0579x; 1.0579x over previous
import jax
import jax.numpy as jnp
from jax import lax
from jax.experimental import pallas as pl
from jax.experimental.pallas import tpu as pltpu

WORD = 256
EMB = 16
H = WORD * EMB
T = 256
NB = H // T
KS = 2
HK = H // KS


WPC = HK // EMB


def _lstm_body(*refs):
    idx_ref, table_ref = refs[0], refs[1]
    w_refs = refs[2:2 + 3 * KS]
    bi_ih, bg_ih, bo_ih, bi_hh, bg_hh, bo_hh = refs[2 + 3 * KS:8 + 3 * KS]
    h1_ref, h2_ref, c_ref, x_s = refs[8 + 3 * KS:]
    j = pl.program_id(0)

    @pl.when(j == 0)
    def _():
        v_iota = lax.broadcasted_iota(jnp.int32, (WORD, WORD), 0)
        onehot_t = (v_iota == idx_ref[...][None, :]).astype(jnp.float32)
        emb = lax.dot_general(
            onehot_t, table_ref[...], (((0,), (0,)), ((), ())),
            preferred_element_type=jnp.float32)
        sel = (lax.broadcasted_iota(jnp.int32, (EMB, HK), 0)
               == lax.broadcasted_iota(jnp.int32, (EMB, HK), 1) % EMB
               ).astype(jnp.float32)
        mask = (lax.broadcasted_iota(jnp.int32, (WPC, HK), 0)
                == lax.broadcasted_iota(jnp.int32, (WPC, HK), 1) // EMB
                ).astype(jnp.float32)
        ones = jnp.ones((1, WPC), jnp.float32)
        for ks in range(KS):
            g = lax.dot_general(
                emb[ks * WPC:(ks + 1) * WPC, :], sel,
                (((1,), (0,)), ((), ())),
                preferred_element_type=jnp.float32)
            xc = lax.dot_general(
                ones, g * mask, (((1,), (0,)), ((), ())),
                preferred_element_type=jnp.float32)
            x_s[0, ks * HK:(ks + 1) * HK] = xc[0, :]

    x = x_s[...]
    dn = (((1,), (1,)), ((), ()))

    def bias(ref):
        return ref[...].reshape(1, T)

    def mv(gate):
        acc = None
        for ks in range(KS):
            part = lax.dot_general(
                x[:, ks * HK:(ks + 1) * HK], w_refs[gate * KS + ks][...],
                dn, preferred_element_type=jnp.float32)
            acc = part if acc is None else acc + part
        return acc

    gi = mv(0) + bias(bi_ih) + bias(bi_hh)
    gg = mv(1) + bias(bg_ih) + bias(bg_hh)
    go = mv(2) + bias(bo_ih) + bias(bo_hh)
    i = jax.nn.sigmoid(gi)
    g = jnp.tanh(gg)
    o = jax.nn.sigmoid(go)
    c = i * g
    h = o * jnp.tanh(c)
    h1_ref[...] = h
    h2_ref[...] = h
    c_ref[...] = c


def _lstm_pallas(idx, table, W_ih, b_ih1, b_hh1):
    w_spec = lambda off, ks: pl.BlockSpec(
        (T, HK), lambda j, off=off, ks=ks: (j + off, ks))
    b_spec = lambda off: pl.BlockSpec((T,), lambda j, off=off: (j + off,))
    in_specs = [
        pl.BlockSpec((WORD,), lambda j: (0,)),
        pl.BlockSpec((WORD, EMB), lambda j: (0, 0)),
    ]
    in_specs += [w_spec(off, ks)
                 for off in (0, 2 * NB, 3 * NB) for ks in range(KS)]
    in_specs += [b_spec(0), b_spec(2 * NB), b_spec(3 * NB)] * 2
    out_specs = [pl.BlockSpec((1, T), lambda j: (0, j))] * 3
    out_shape = [jax.ShapeDtypeStruct((1, H), jnp.float32)] * 3
    return pl.pallas_call(
        _lstm_body,
        grid=(NB,),
        in_specs=in_specs,
        out_specs=out_specs,
        out_shape=out_shape,
        scratch_shapes=[pltpu.VMEM((1, H), jnp.float32)],
    )(idx, table, *([W_ih] * (3 * KS)),
      b_ih1, b_ih1, b_ih1, b_hh1, b_hh1, b_hh1)


def kernel(input, table, W_ih, W_hh, b_ih, b_hh):
    del W_hh
    h1, h2, c = _lstm_pallas(input.astype(jnp.int32), table, W_ih,
                             b_ih, b_hh)
    return (h1.reshape(1, 1, H), h2.reshape(1, 1, H), c.reshape(1, 1, H))
